# baseline (speedup 1.0000x reference)
import functools
import math

import jax
import jax.numpy as jnp
from jax.experimental import pallas as pl
from jax.experimental.pallas import tpu as pltpu

_NUM_NEIGHBORS = 1.5
_N_BLOCK = 1000


def _gate_body(v_ref, o_ref, scale):
    v = v_ref[...] * scale
    s_e = jax.nn.gelu(v[:, 0:10])
    s_o = jnp.tanh(v[:, 10:20])
    g = jax.nn.sigmoid(v[:, 20:30])
    vec = v[:, 30:60].reshape(-1, 10, 3) * g[:, :, None]
    o_ref[...] = jnp.concatenate([s_e, s_o, vec.reshape(-1, 30)], axis=1)


def _gate(v, scale):
    n = v.shape[0]
    return pl.pallas_call(
        functools.partial(_gate_body, scale=scale),
        grid=(n // _N_BLOCK,),
        in_specs=[pl.BlockSpec((_N_BLOCK, 60), lambda i: (i, 0))],
        out_specs=pl.BlockSpec((_N_BLOCK, 50), lambda i: (i, 0)),
        out_shape=jax.ShapeDtypeStruct((n, 50), jnp.float32),
    )(v)


def _conv(x, edge_src, edge_dst, edge_attr, W, scale=1.0):
    xs = jnp.take(x, edge_src, axis=0)
    msg = jnp.zeros((edge_src.shape[0], W.shape[0]), dtype=x.dtype)
    for j in range(edge_attr.shape[1]):
        msg = msg + (xs @ W[:, :, j].T) * edge_attr[:, j][:, None]
    out = jax.ops.segment_sum(msg, edge_dst, num_segments=x.shape[0])
    return out * scale


def kernel(x, edge_src, edge_dst, edge_attr, W0, W1, W2, W3, W4):
    inv = float(1.0 / math.sqrt(_NUM_NEIGHBORS))
    h = _gate(_conv(x, edge_src, edge_dst, edge_attr, W0), inv)
    for W in (W1, W2, W3):
        h = _gate(_conv(h, edge_src, edge_dst, edge_attr, W), inv)
    return _conv(h, edge_src, edge_dst, edge_attr, W4, scale=inv)


# speedup vs baseline: 2.1170x; 2.1170x over previous
import functools
import math

import jax
import jax.numpy as jnp
from jax import lax
from jax.experimental import pallas as pl
from jax.experimental.pallas import tpu as pltpu
from jax.experimental.pallas import tpu_sc as plsc

_INV_SQRT_NN = float(1.0 / math.sqrt(1.5))
_NC, _NS = 2, 16
_NW = _NC * _NS
_KW = 1000
_KWS = 400
_D = 64

_SC_PARAMS = pltpu.CompilerParams(use_tc_tiling_on_sc=False)


def _mesh():
    return plsc.VectorSubcoreMesh(core_axis_name="c", subcore_axis_name="s")


def _make_gather(E, N):
    epw = E // _NW
    nwin = epw // _KW

    @functools.partial(
        pl.kernel,
        out_type=jax.ShapeDtypeStruct((E, _D), jnp.float32),
        mesh=_mesh(),
        compiler_params=_SC_PARAMS,
        scratch_types=[
            pltpu.VMEM((_KW,), jnp.int32),
            pltpu.VMEM((_KW, _D), jnp.float32),
            pltpu.SemaphoreType.DMA,
        ],
    )
    def gather_k(h_hbm, src_hbm, xs_hbm, idx_v, rows_v, sem):
        wid = lax.axis_index("s") * _NC + lax.axis_index("c")
        base = wid * epw

        def body(w, carry):
            off = base + w * _KW
            pltpu.sync_copy(src_hbm.at[pl.ds(off, _KW)], idx_v)
            pltpu.async_copy(h_hbm.at[idx_v], rows_v, sem).wait()
            pltpu.sync_copy(rows_v, xs_hbm.at[pl.ds(off, _KW)])
            return carry

        lax.fori_loop(0, nwin, body, 0)

    return gather_k


def _make_scatter(E, N):
    C = 32
    epw = E // _NS
    nwin = epw // _KWS
    rows_pt = N // _NS

    @functools.partial(
        pl.kernel,
        out_type=jax.ShapeDtypeStruct((2, N, C), jnp.float32),
        mesh=_mesh(),
        compiler_params=_SC_PARAMS,
        scratch_types=[
            pltpu.VMEM_SHARED((N, C), jnp.float32),
            pltpu.VMEM((_KWS,), jnp.int32),
            pltpu.VMEM((_KWS, C), jnp.float32),
        ],
    )
    def scatter_k(msg_hbm, dst_hbm, zeros_hbm, out_hbm, acc_s, idx_v, buf_v):
        c = lax.axis_index("c")
        s = lax.axis_index("s")
        nbase = s * rows_pt
        pltpu.sync_copy(zeros_hbm.at[pl.ds(0, rows_pt)],
                        acc_s.at[pl.ds(nbase, rows_pt)])
        plsc.subcore_barrier()
        base = s * epw

        def body(w, carry):
            off = base + w * _KWS
            pltpu.sync_copy(dst_hbm.at[pl.ds(off, _KWS)], idx_v)
            pltpu.sync_copy(msg_hbm.at[pl.ds(off, _KWS), pl.ds(c * C, C)],
                            buf_v)
            pltpu.sync_copy(buf_v, acc_s.at[idx_v], add=True)
            return carry

        lax.fori_loop(0, nwin, body, 0)
        plsc.subcore_barrier()
        pltpu.sync_copy(acc_s.at[pl.ds(nbase, rows_pt)],
                        out_hbm.at[c, pl.ds(nbase, rows_pt)])

    return scatter_k


def _tp_body(xs_ref, attr_ref, w_ref, o_ref):
    bt = attr_ref.shape[0]
    xsp = xs_ref[...].reshape(bt, 2 * _D)
    a = attr_ref[...]
    acc = jnp.zeros((bt, 2 * _D), jnp.float32)
    for j in range(4):
        apj = jnp.concatenate(
            [jnp.broadcast_to(a[:, j:j + 1], (bt, _D)),
             jnp.broadcast_to(a[:, 4 + j:5 + j], (bt, _D))], axis=1)
        acc = acc + jnp.dot(xsp, w_ref[j],
                            preferred_element_type=jnp.float32) * apj
    o_ref[...] = acc.reshape(bt * 2 * _D)


def _tp(xs_flat, attr_pairs, wp, Et=2000):
    Ef = xs_flat.shape[0]
    E = Ef // _D
    grid = (E // Et,)
    return pl.pallas_call(
        _tp_body,
        grid=grid,
        in_specs=[
            pl.BlockSpec((Et * _D,), lambda i: (i,)),
            pl.BlockSpec((Et // 2, 8), lambda i: (i, 0)),
            pl.BlockSpec((4, 2 * _D, 2 * _D), lambda i: (0, 0, 0)),
        ],
        out_specs=pl.BlockSpec((Et * _D,), lambda i: (i,)),
        out_shape=jax.ShapeDtypeStruct((Ef,), jnp.float32),
    )(xs_flat, attr_pairs, wp)


def _gate_mid_body(a_ref, b_ref, o_ref):
    v = jnp.concatenate([a_ref[...], b_ref[:, :28]], axis=1) * _INV_SQRT_NN
    s_e = jax.nn.gelu(v[:, 0:10])
    s_o = jnp.tanh(v[:, 10:20])
    g = jax.nn.sigmoid(v[:, 20:30])
    vec = (v[:, 30:60].reshape(-1, 10, 3) * g[:, :, None]).reshape(-1, 30)
    z = jnp.zeros((v.shape[0], 14), jnp.float32)
    o_ref[...] = jnp.concatenate([s_e, s_o, vec, z], axis=1)


def _gate_l4_body(a_ref, b_ref, o_ref):
    o_ref[...] = (a_ref[:, :7] + b_ref[:, :7]) * _INV_SQRT_NN


def _gate_mid(pa, pb, Nb=2000):
    N = pa.shape[0]
    return pl.pallas_call(
        _gate_mid_body,
        grid=(N // Nb,),
        in_specs=[pl.BlockSpec((Nb, 32), lambda i: (i, 0)),
                  pl.BlockSpec((Nb, 32), lambda i: (i, 0))],
        out_specs=pl.BlockSpec((Nb, _D), lambda i: (i, 0)),
        out_shape=jax.ShapeDtypeStruct((N, _D), jnp.float32),
    )(pa, pb)


def _gate_l4(pa, pb, Nb=2000):
    N = pa.shape[0]
    return pl.pallas_call(
        _gate_l4_body,
        grid=(N // Nb,),
        in_specs=[pl.BlockSpec((Nb, 32), lambda i: (i, 0)),
                  pl.BlockSpec((Nb, 32), lambda i: (i, 0))],
        out_specs=pl.BlockSpec((Nb, 7), lambda i: (i, 0)),
        out_shape=jax.ShapeDtypeStruct((N, 7), jnp.float32),
    )(pa, pb)


def _pack_w(W):
    dout, din, _ = W.shape
    wj = jnp.transpose(W, (2, 1, 0))
    wj = jnp.pad(wj, ((0, 0), (0, _D - din), (0, _D - dout)))
    z = jnp.zeros((4, _D, _D), jnp.float32)
    top = jnp.concatenate([wj, z], axis=2)
    bot = jnp.concatenate([z, wj], axis=2)
    return jnp.concatenate([top, bot], axis=1)


def kernel(x, edge_src, edge_dst, edge_attr, W0, W1, W2, W3, W4):
    N = x.shape[0]
    E = edge_src.shape[0]
    src = edge_src.astype(jnp.int32)
    dst = edge_dst.astype(jnp.int32)
    attr_pairs = edge_attr.reshape(E // 2, 8)
    zeros32 = jnp.zeros((N // _NS, 32), jnp.float32)

    gather = _make_gather(E, N)
    scatter = _make_scatter(E, N)

    h = jnp.pad(x, ((0, 0), (0, _D - 1)))
    for li, W in enumerate((W0, W1, W2, W3, W4)):
        wp = _pack_w(W)
        xs = gather(h, src)
        din = W.shape[1]
        wj = wp[:, :din, :60]
        tp = jnp.zeros((E, 60), jnp.float32)
        for j in range(4):
            tp = tp + (xs[:, :din] @ wj[j]) * edge_attr[:, j][:, None]
        msg = jnp.pad(tp, ((0, 0), (0, 4)))
        parts = scatter(msg, dst, zeros32)
        if li < 4:
            h = _gate_mid(parts[0], parts[1])
        else:
            return _gate_l4(parts[0], parts[1])


# speedup vs baseline: 3.9974x; 1.8883x over previous
import functools
import math

import jax
import jax.numpy as jnp
from jax import lax
from jax.experimental import pallas as pl
from jax.experimental.pallas import tpu as pltpu
from jax.experimental.pallas import tpu_sc as plsc

_INV_SQRT_NN = float(1.0 / math.sqrt(1.5))
_NC, _NS = 2, 16
_NW = _NC * _NS
_KW = 1000
_KWS = 400
_D = 64

_SC_PARAMS = pltpu.CompilerParams(use_tc_tiling_on_sc=False)


def _mesh():
    return plsc.VectorSubcoreMesh(core_axis_name="c", subcore_axis_name="s")


def _make_gather(E, N):
    epw = E // _NW
    nwin = epw // _KW

    @functools.partial(
        pl.kernel,
        out_type=jax.ShapeDtypeStruct((E, _D), jnp.float32),
        mesh=_mesh(),
        compiler_params=_SC_PARAMS,
        scratch_types=[
            pltpu.VMEM((_KW,), jnp.int32),
            pltpu.VMEM((_KW, _D), jnp.float32),
            pltpu.SemaphoreType.DMA,
        ],
    )
    def gather_k(h_hbm, src_hbm, xs_hbm, idx_v, rows_v, sem):
        wid = lax.axis_index("s") * _NC + lax.axis_index("c")
        base = wid * epw

        def body(w, carry):
            off = base + w * _KW
            pltpu.sync_copy(src_hbm.at[pl.ds(off, _KW)], idx_v)
            pltpu.async_copy(h_hbm.at[idx_v], rows_v, sem).wait()
            pltpu.sync_copy(rows_v, xs_hbm.at[pl.ds(off, _KW)])
            return carry

        lax.fori_loop(0, nwin, body, 0)

    return gather_k


def _make_scatter(E, N):
    C = 32
    epw = E // _NS
    nwin = epw // _KWS
    rows_pt = N // _NS

    @functools.partial(
        pl.kernel,
        out_type=jax.ShapeDtypeStruct((2, N, C), jnp.float32),
        mesh=_mesh(),
        compiler_params=_SC_PARAMS,
        scratch_types=[
            pltpu.VMEM_SHARED((N, C), jnp.float32),
            pltpu.VMEM((_KWS,), jnp.int32),
            pltpu.VMEM((_KWS, C), jnp.float32),
        ],
    )
    def scatter_k(msg_hbm, dst_hbm, zeros_hbm, out_hbm, acc_s, idx_v, buf_v):
        c = lax.axis_index("c")
        s = lax.axis_index("s")
        nbase = s * rows_pt
        pltpu.sync_copy(zeros_hbm.at[pl.ds(0, rows_pt)],
                        acc_s.at[pl.ds(nbase, rows_pt)])
        plsc.subcore_barrier()
        base = s * epw

        def body(w, carry):
            off = base + w * _KWS
            pltpu.sync_copy(dst_hbm.at[pl.ds(off, _KWS)], idx_v)
            pltpu.sync_copy(msg_hbm.at[pl.ds(off, _KWS), pl.ds(c * C, C)],
                            buf_v)
            pltpu.sync_copy(buf_v, acc_s.at[idx_v], add=True)
            return carry

        lax.fori_loop(0, nwin, body, 0)
        plsc.subcore_barrier()
        pltpu.sync_copy(acc_s.at[pl.ds(nbase, rows_pt)],
                        out_hbm.at[c, pl.ds(nbase, rows_pt)])

    return scatter_k


def _tp_body(xs_ref, attr_ref, w_ref, o_ref, *, exact_k1):
    bt = attr_ref.shape[0]
    xsp = xs_ref[...].reshape(bt, 2 * _D)
    a = attr_ref[...]
    acc = jnp.zeros((bt, 2 * _D), jnp.float32)
    for j in range(4):
        apj = jnp.concatenate(
            [jnp.broadcast_to(a[:, j:j + 1], (bt, _D)),
             jnp.broadcast_to(a[:, 4 + j:5 + j], (bt, _D))], axis=1)
        if exact_k1:
            wrow = w_ref[j, 0, 0:_D]
            prod = jnp.concatenate(
                [xsp[:, 0:1] * wrow[None, :],
                 xsp[:, _D:_D + 1] * wrow[None, :]], axis=1)
        else:
            prod = jnp.dot(xsp, w_ref[j],
                           preferred_element_type=jnp.float32)
        acc = acc + prod * apj
    o_ref[...] = acc.reshape(bt * 2 * _D)


def _tp(xs_flat, attr_pairs, wp, exact_k1=False, Et=2000):
    Ef = xs_flat.shape[0]
    E = Ef // _D
    grid = (E // Et,)
    return pl.pallas_call(
        functools.partial(_tp_body, exact_k1=exact_k1),
        grid=grid,
        in_specs=[
            pl.BlockSpec((Et * _D,), lambda i: (i,)),
            pl.BlockSpec((Et // 2, 8), lambda i: (i, 0)),
            pl.BlockSpec((4, 2 * _D, 2 * _D), lambda i: (0, 0, 0)),
        ],
        out_specs=pl.BlockSpec((Et * _D,), lambda i: (i,)),
        out_shape=jax.ShapeDtypeStruct((Ef,), jnp.float32),
    )(xs_flat, attr_pairs, wp)


def _gate_mid_body(a_ref, b_ref, o_ref):
    v = jnp.concatenate([a_ref[...], b_ref[:, :28]], axis=1) * _INV_SQRT_NN
    s_e = jax.nn.gelu(v[:, 0:10])
    s_o = jnp.tanh(v[:, 10:20])
    g = jax.nn.sigmoid(v[:, 20:30])
    vec = (v[:, 30:60].reshape(-1, 10, 3) * g[:, :, None]).reshape(-1, 30)
    z = jnp.zeros((v.shape[0], 14), jnp.float32)
    o_ref[...] = jnp.concatenate([s_e, s_o, vec, z], axis=1)


def _gate_l4_body(a_ref, b_ref, o_ref):
    o_ref[...] = (a_ref[:, :7] + b_ref[:, :7]) * _INV_SQRT_NN


def _gate_mid(pa, pb, Nb=2000):
    N = pa.shape[0]
    return pl.pallas_call(
        _gate_mid_body,
        grid=(N // Nb,),
        in_specs=[pl.BlockSpec((Nb, 32), lambda i: (i, 0)),
                  pl.BlockSpec((Nb, 32), lambda i: (i, 0))],
        out_specs=pl.BlockSpec((Nb, _D), lambda i: (i, 0)),
        out_shape=jax.ShapeDtypeStruct((N, _D), jnp.float32),
    )(pa, pb)


def _gate_l4(pa, pb, Nb=2000):
    N = pa.shape[0]
    return pl.pallas_call(
        _gate_l4_body,
        grid=(N // Nb,),
        in_specs=[pl.BlockSpec((Nb, 32), lambda i: (i, 0)),
                  pl.BlockSpec((Nb, 32), lambda i: (i, 0))],
        out_specs=pl.BlockSpec((Nb, 7), lambda i: (i, 0)),
        out_shape=jax.ShapeDtypeStruct((N, 7), jnp.float32),
    )(pa, pb)


def _pack_w(W):
    dout, din, _ = W.shape
    wj = jnp.transpose(W, (2, 1, 0))
    wj = jnp.pad(wj, ((0, 0), (0, _D - din), (0, _D - dout)))
    z = jnp.zeros((4, _D, _D), jnp.float32)
    top = jnp.concatenate([wj, z], axis=2)
    bot = jnp.concatenate([z, wj], axis=2)
    return jnp.concatenate([top, bot], axis=1)


def kernel(x, edge_src, edge_dst, edge_attr, W0, W1, W2, W3, W4):
    N = x.shape[0]
    E = edge_src.shape[0]
    src = edge_src.astype(jnp.int32)
    dst = edge_dst.astype(jnp.int32)
    attr_pairs = edge_attr.reshape(E // 2, 8)
    zeros32 = jnp.zeros((N // _NS, 32), jnp.float32)

    gather = _make_gather(E, N)
    scatter = _make_scatter(E, N)

    h = jnp.pad(x, ((0, 0), (0, _D - 1)))
    for li, W in enumerate((W0, W1, W2, W3, W4)):
        wp = _pack_w(W)
        xs = gather(h, src)
        msg = _tp(xs.reshape(E * _D), attr_pairs, wp,
                  exact_k1=(li == 0)).reshape(E, _D)
        parts = scatter(msg, dst, zeros32)
        if li < 4:
            h = _gate_mid(parts[0], parts[1])
        else:
            return _gate_l4(parts[0], parts[1])
